# Initial kernel scaffold; baseline (speedup 1.0000x reference)
import jax
import jax.numpy as jnp
from jax.experimental import pallas as pl


def kernel(frame_t, frame_R, edge_src, edge_dst, W):
    raise NotImplementedError("write your pallas kernel here")


# speedup vs baseline: 9.8470x; 9.8470x over previous
import functools

import jax
import jax.numpy as jnp
from jax import lax
from jax.experimental import pallas as pl
from jax.experimental.pallas import tpu as pltpu
from jax.experimental.pallas import tpu_sc as plsc

_N_NODES = 100000
_N_EDGES = 1600000
_N_BASIS = 16
_OUT_DIM = 32
_D_MAX = 20.0

_NC = 2
_NS = 16
_NW = _NC * _NS
_L = 16

_BLK = 80
_EPW = _N_EDGES // _NW
_NBLK = _EPW // _BLK
_GRP = _BLK // _L


def _rsqrt(a):
    i = plsc.bitcast(a, jnp.int32)
    y = plsc.bitcast(jnp.int32(0x5F3759DF) - (i >> 1), jnp.float32)
    h = 0.5 * a
    for _ in range(2):
        y = y * (1.5 - h * y * y)
    return y


def _sc_body(table, esrc, edst, feats, isv, idv, rs, rd, fb, s1, s2):
    cid = lax.axis_index("c")
    sid = lax.axis_index("s")
    wid = sid * _NC + cid
    ii = lax.iota(jnp.int32, _L)
    centers = [jnp.float32(_D_MAX * b / (_N_BASIS - 1)) for b in range(_N_BASIS)]
    zero = jnp.zeros((_L,), jnp.float32)
    cols = [jnp.full((_L,), f, jnp.int32) for f in range(_OUT_DIM)]

    def block(i, carry):
        ebase = wid * _EPW + i * _BLK
        pltpu.sync_copy(esrc.at[pl.ds(ebase, _BLK)], isv)
        pltpu.sync_copy(edst.at[pl.ds(ebase, _BLK)], idv)
        cps = pltpu.async_copy(table.at[isv], rs, s1)
        cpd = pltpu.async_copy(table.at[idv], rd, s2)
        cps.wait()
        cpd.wait()
        for g in range(_GRP):
            rows = g * _L + ii
            s = [plsc.load_gather(rs, [rows, cols[f]]) for f in range(12)]
            d = [plsc.load_gather(rd, [rows, cols[f]]) for f in range(12)]
            dx = d[0] - s[0]
            dy = d[1] - s[1]
            dz = d[2] - s[2]
            d2 = dx * dx + dy * dy + dz * dz
            dist = d2 * _rsqrt(jnp.maximum(d2, jnp.float32(1e-20)))
            inv_den = _rsqrt(d2 + 1.0)
            Rs = s[3:]
            Rd = d[3:]
            fv = []
            for c in centers:
                t = dist - c
                fv.append(jnp.exp(-(t * t)))
            for k in range(3):
                fv.append((dx * Rs[k] + dy * Rs[3 + k] + dz * Rs[6 + k]) * inv_den)
            ninv = -inv_den
            for k in range(3):
                fv.append((dx * Rd[k] + dy * Rd[3 + k] + dz * Rd[6 + k]) * ninv)
            for a in range(3):
                for k in range(3):
                    fv.append(Rs[a] * Rd[k] + Rs[3 + a] * Rd[3 + k]
                              + Rs[6 + a] * Rd[6 + k])
            fv.append(zero)
            for f, v in enumerate(fv):
                plsc.store_scatter(fb, [rows, cols[f]], v)
        pltpu.sync_copy(fb, feats.at[pl.ds(ebase, _BLK)])
        return carry

    lax.fori_loop(0, _NBLK, block, 0)


_sc_feats = functools.partial(
    pl.kernel,
    out_type=jax.ShapeDtypeStruct((_N_EDGES, _OUT_DIM), jnp.float32),
    mesh=plsc.VectorSubcoreMesh(core_axis_name="c", subcore_axis_name="s",
                                num_cores=_NC, num_subcores=_NS),
    scratch_types=[
        pltpu.VMEM((_BLK,), jnp.int32),
        pltpu.VMEM((_BLK,), jnp.int32),
        pltpu.VMEM((_BLK, 16), jnp.float32),
        pltpu.VMEM((_BLK, 16), jnp.float32),
        pltpu.VMEM((_BLK, _OUT_DIM), jnp.float32),
        pltpu.SemaphoreType.DMA,
        pltpu.SemaphoreType.DMA,
    ],
    compiler_params=pltpu.CompilerParams(use_tc_tiling_on_sc=False,
                                         needs_layout_passes=False),
)(_sc_body)

_TBE = 6400


def _mm_body(x_ref, w_ref, o_ref):
    o_ref[...] = jnp.dot(x_ref[...], w_ref[...],
                         preferred_element_type=jnp.float32)


def _matmul(feats, w_pad):
    return pl.pallas_call(
        _mm_body,
        grid=(_N_EDGES // _TBE,),
        in_specs=[pl.BlockSpec((_TBE, _OUT_DIM), lambda i: (i, 0)),
                  pl.BlockSpec((_OUT_DIM, _OUT_DIM), lambda i: (0, 0))],
        out_specs=pl.BlockSpec((_TBE, _OUT_DIM), lambda i: (i, 0)),
        out_shape=jax.ShapeDtypeStruct((_N_EDGES, _OUT_DIM), jnp.float32),
    )(feats, w_pad)


def kernel(frame_t, frame_R, edge_src, edge_dst, W):
    table = jnp.concatenate(
        [frame_t, frame_R.reshape(_N_NODES, 9),
         jnp.zeros((_N_NODES, 4), jnp.float32)], axis=1)
    w_pad = jnp.zeros((_OUT_DIM, _OUT_DIM), jnp.float32).at[:31, :].set(W)
    feats = _sc_feats(table, edge_src.astype(jnp.int32),
                      edge_dst.astype(jnp.int32))
    return _matmul(feats, w_pad)


# speedup vs baseline: 13.7926x; 1.4007x over previous
import functools

import jax
import jax.numpy as jnp
from jax import lax
from jax.experimental import pallas as pl
from jax.experimental.pallas import tpu as pltpu
from jax.experimental.pallas import tpu_sc as plsc

_N_NODES = 100000
_N_EDGES = 1600000
_N_BASIS = 16
_OUT_DIM = 32
_D_MAX = 20.0

_NC = 2
_NS = 16
_NW = _NC * _NS
_L = 16

_BLK = 400
_SUB = 80
_NSUB = _BLK // _SUB
_EPW = _N_EDGES // _NW
_NBLKW = _EPW // _BLK
_GRP = _BLK // _L


def _rsqrt(a):
    i = plsc.bitcast(a, jnp.int32)
    y = plsc.bitcast(jnp.int32(0x5F3759DF) - (i >> 1), jnp.float32)
    h = 0.5 * a
    for _ in range(2):
        y = y * (1.5 - h * y * y)
    return y


def _sc_body(table, esrc, edst, feats,
             isv0, idv0, isv1, idv1,
             rs0, rd0, rs1, rd1,
             fb0, fb1,
             s_idx0, s_idx1, sgs0, sgd0, sgs1, sgd1, so0, so1):
    cid = lax.axis_index("c")
    sid = lax.axis_index("s")
    wid = sid * _NC + cid
    wbase = wid * _EPW
    ii = lax.iota(jnp.int32, _L)
    centers = [jnp.float32(_D_MAX * b / (_N_BASIS - 1)) for b in range(_N_BASIS)]
    zero = jnp.zeros((_L,), jnp.float32)
    cols = [jnp.full((_L,), f, jnp.int32) for f in range(_OUT_DIM)]

    def fire_idx(iref, dref, sem, blk):
        ebase = wbase + blk * _BLK
        pltpu.async_copy(esrc.at[pl.ds(ebase, _BLK)], iref, sem)
        pltpu.async_copy(edst.at[pl.ds(ebase, _BLK)], dref, sem)

    def wait_idx(iref, dref, sem):
        pltpu.make_async_copy(esrc.at[pl.ds(0, _BLK)], iref, sem).wait()
        pltpu.make_async_copy(esrc.at[pl.ds(0, _BLK)], dref, sem).wait()

    def fire_gathers(iref, dref, rs, rd, sgs, sgd):
        for k in range(_NSUB):
            sl = pl.ds(k * _SUB, _SUB)
            pltpu.async_copy(table.at[iref.at[sl]], rs.at[sl], sgs)
            pltpu.async_copy(table.at[dref.at[sl]], rd.at[sl], sgd)

    def wait_gathers(rs, rd, sgs, sgd):
        pltpu.make_async_copy(table.at[pl.ds(0, _BLK)], rs, sgs).wait()
        pltpu.make_async_copy(table.at[pl.ds(0, _BLK)], rd, sgd).wait()

    def fire_out(fb, sem, blk):
        pltpu.async_copy(fb, feats.at[pl.ds(wbase + blk * _BLK, _BLK)], sem)

    def wait_out(fb, sem):
        pltpu.make_async_copy(fb, feats.at[pl.ds(0, _BLK)], sem).wait()

    def compute_block(rs, rd, fb):
        def grp(g, carry):
            rows = g * _L + ii
            s = [plsc.load_gather(rs, [rows, cols[f]]) for f in range(12)]
            d = [plsc.load_gather(rd, [rows, cols[f]]) for f in range(12)]
            dx = d[0] - s[0]
            dy = d[1] - s[1]
            dz = d[2] - s[2]
            d2 = dx * dx + dy * dy + dz * dz
            dist = d2 * _rsqrt(jnp.maximum(d2, jnp.float32(1e-20)))
            inv_den = _rsqrt(d2 + 1.0)
            Rs = s[3:]
            Rd = d[3:]
            fv = []
            for c in centers:
                t = dist - c
                fv.append(jnp.exp(-(t * t)))
            for k in range(3):
                fv.append((dx * Rs[k] + dy * Rs[3 + k] + dz * Rs[6 + k]) * inv_den)
            ninv = -inv_den
            for k in range(3):
                fv.append((dx * Rd[k] + dy * Rd[3 + k] + dz * Rd[6 + k]) * ninv)
            for a in range(3):
                for k in range(3):
                    fv.append(Rs[a] * Rd[k] + Rs[3 + a] * Rd[3 + k]
                              + Rs[6 + a] * Rd[6 + k])
            fv.append(zero)
            for f, v in enumerate(fv):
                plsc.store_scatter(fb, [rows, cols[f]], v)
            return carry

        lax.fori_loop(0, _GRP, grp, 0)

    fire_idx(isv0, idv0, s_idx0, 0)
    fire_idx(isv1, idv1, s_idx1, 1)
    wait_idx(isv0, idv0, s_idx0)
    fire_gathers(isv0, idv0, rs0, rd0, sgs0, sgd0)

    def pair(k, carry):
        a = 2 * k
        b = a + 1
        wait_idx(isv1, idv1, s_idx1)
        fire_gathers(isv1, idv1, rs1, rd1, sgs1, sgd1)
        wait_gathers(rs0, rd0, sgs0, sgd0)
        fire_idx(isv0, idv0, s_idx0, a + 2)

        @pl.when(k >= 1)
        def _():
            wait_out(fb0, so0)

        compute_block(rs0, rd0, fb0)
        fire_out(fb0, so0, a)

        wait_idx(isv0, idv0, s_idx0)
        fire_gathers(isv0, idv0, rs0, rd0, sgs0, sgd0)
        wait_gathers(rs1, rd1, sgs1, sgd1)

        @pl.when(b + 2 <= _NBLKW - 1)
        def _():
            fire_idx(isv1, idv1, s_idx1, b + 2)

        @pl.when(k >= 1)
        def _():
            wait_out(fb1, so1)

        compute_block(rs1, rd1, fb1)
        fire_out(fb1, so1, b)
        return carry

    lax.fori_loop(0, (_NBLKW - 1) // 2, pair, 0)

    wait_gathers(rs0, rd0, sgs0, sgd0)
    wait_out(fb0, so0)
    compute_block(rs0, rd0, fb0)
    fire_out(fb0, so0, _NBLKW - 1)
    wait_out(fb1, so1)
    wait_out(fb0, so0)


_sc_feats = functools.partial(
    pl.kernel,
    out_type=jax.ShapeDtypeStruct((_N_EDGES, _OUT_DIM), jnp.float32),
    mesh=plsc.VectorSubcoreMesh(core_axis_name="c", subcore_axis_name="s",
                                num_cores=_NC, num_subcores=_NS),
    scratch_types=[
        pltpu.VMEM((_BLK,), jnp.int32),
        pltpu.VMEM((_BLK,), jnp.int32),
        pltpu.VMEM((_BLK,), jnp.int32),
        pltpu.VMEM((_BLK,), jnp.int32),
        pltpu.VMEM((_BLK, 16), jnp.float32),
        pltpu.VMEM((_BLK, 16), jnp.float32),
        pltpu.VMEM((_BLK, 16), jnp.float32),
        pltpu.VMEM((_BLK, 16), jnp.float32),
        pltpu.VMEM((_BLK, _OUT_DIM), jnp.float32),
        pltpu.VMEM((_BLK, _OUT_DIM), jnp.float32),
        pltpu.SemaphoreType.DMA,
        pltpu.SemaphoreType.DMA,
        pltpu.SemaphoreType.DMA,
        pltpu.SemaphoreType.DMA,
        pltpu.SemaphoreType.DMA,
        pltpu.SemaphoreType.DMA,
        pltpu.SemaphoreType.DMA,
        pltpu.SemaphoreType.DMA,
    ],
    compiler_params=pltpu.CompilerParams(use_tc_tiling_on_sc=False,
                                         needs_layout_passes=False),
)(_sc_body)

_TBE = 6400


def _mm_body(x_ref, w_ref, o_ref):
    o_ref[...] = jnp.dot(x_ref[...], w_ref[...],
                         preferred_element_type=jnp.float32)


def _matmul(feats, w_pad):
    return pl.pallas_call(
        _mm_body,
        grid=(_N_EDGES // _TBE,),
        in_specs=[pl.BlockSpec((_TBE, _OUT_DIM), lambda i: (i, 0)),
                  pl.BlockSpec((_OUT_DIM, _OUT_DIM), lambda i: (0, 0))],
        out_specs=pl.BlockSpec((_TBE, _OUT_DIM), lambda i: (i, 0)),
        out_shape=jax.ShapeDtypeStruct((_N_EDGES, _OUT_DIM), jnp.float32),
    )(feats, w_pad)


def kernel(frame_t, frame_R, edge_src, edge_dst, W):
    table = jnp.concatenate(
        [frame_t, frame_R.reshape(_N_NODES, 9),
         jnp.zeros((_N_NODES, 4), jnp.float32)], axis=1)
    w_pad = jnp.zeros((_OUT_DIM, _OUT_DIM), jnp.float32).at[:31, :].set(W)
    feats = _sc_feats(table, edge_src.astype(jnp.int32),
                      edge_dst.astype(jnp.int32))
    return _matmul(feats, w_pad)


# speedup vs baseline: 18.0954x; 1.3120x over previous
import functools

import jax
import jax.numpy as jnp
from jax import lax
from jax.experimental import pallas as pl
from jax.experimental.pallas import tpu as pltpu
from jax.experimental.pallas import tpu_sc as plsc

_N_NODES = 100000
_N_EDGES = 1600000
_N_BASIS = 16
_OUT_DIM = 32
_D_MAX = 20.0

_NC = 2
_NS = 16
_NW = _NC * _NS
_L = 16

_BLK = 400
_SUB = 80
_NSUB = _BLK // _SUB
_GRP = _BLK // _L

_SBE = 6400
_NSB = _N_EDGES // _SBE
_TBS = _SBE // 4
_SB8 = _NSB - 7 * _NW


def _rsqrt(a):
    i = plsc.bitcast(a, jnp.int32)
    y = plsc.bitcast(jnp.int32(0x5F3759DF) - (i >> 1), jnp.float32)
    h = 0.5 * a
    for _ in range(2):
        y = y * (1.5 - h * y * y)
    return y


def _sc_body(table, esrc, edst, feats,
             isv0, idv0, isv1, idv1,
             rs0, rd0, rs1, rd1,
             fb0, fb1,
             s_idx0, s_idx1, sgs0, sgd0, sgs1, sgd1, so0, so1):
    cid = lax.axis_index("c")
    sid = lax.axis_index("s")
    wid = sid * _NC + cid
    sb_start = jnp.where(wid < _SB8, 8 * wid, 7 * wid + _SB8)
    nblk = jnp.where(wid < _SB8, 8 * _SBE // _BLK, 7 * _SBE // _BLK)
    npair = nblk // 2
    wbase = sb_start * _SBE
    ii = lax.iota(jnp.int32, _L)
    centers = [jnp.float32(_D_MAX * b / (_N_BASIS - 1)) for b in range(_N_BASIS)]
    zero = jnp.zeros((_L,), jnp.float32)
    cols = [jnp.full((_L,), f, jnp.int32) for f in range(_OUT_DIM)]

    def fire_idx(iref, dref, sem, blk):
        ebase = wbase + blk * _BLK
        pltpu.async_copy(esrc.at[pl.ds(ebase, _BLK)], iref, sem)
        pltpu.async_copy(edst.at[pl.ds(ebase, _BLK)], dref, sem)

    def wait_idx(iref, dref, sem):
        pltpu.make_async_copy(esrc.at[pl.ds(0, _BLK)], iref, sem).wait()
        pltpu.make_async_copy(esrc.at[pl.ds(0, _BLK)], dref, sem).wait()

    def fire_gathers(iref, dref, rs, rd, sgs, sgd):
        for k in range(_NSUB):
            sl = pl.ds(k * _SUB, _SUB)
            pltpu.async_copy(table.at[iref.at[sl]], rs.at[sl], sgs)
            pltpu.async_copy(table.at[dref.at[sl]], rd.at[sl], sgd)

    def wait_gathers(rs, rd, sgs, sgd):
        pltpu.make_async_copy(table.at[pl.ds(0, _BLK)], rs, sgs).wait()
        pltpu.make_async_copy(table.at[pl.ds(0, _BLK)], rd, sgd).wait()

    def fire_out(fb, sem, blk):
        eb = wbase + blk * _BLK
        sb = eb // _SBE
        off = eb % _SBE
        jb = off // _TBS
        r0 = sb * _TBS + off % _TBS
        pltpu.async_copy(
            fb, feats.at[pl.ds(r0, _BLK), pl.ds(jb * 32, 32)], sem)

    def wait_out(fb, sem):
        pltpu.make_async_copy(
            fb, feats.at[pl.ds(0, _BLK), pl.ds(0, 32)], sem).wait()

    def compute_block(rs, rd, fb):
        def grp(g, carry):
            rows = g * _L + ii
            s = [plsc.load_gather(rs, [rows, cols[f]]) for f in range(12)]
            d = [plsc.load_gather(rd, [rows, cols[f]]) for f in range(12)]
            dx = d[0] - s[0]
            dy = d[1] - s[1]
            dz = d[2] - s[2]
            d2 = dx * dx + dy * dy + dz * dz
            dist = d2 * _rsqrt(jnp.maximum(d2, jnp.float32(1e-20)))
            inv_den = _rsqrt(d2 + 1.0)
            Rs = s[3:]
            Rd = d[3:]
            fv = []
            for c in centers:
                t = dist - c
                fv.append(jnp.exp(-(t * t)))
            for k in range(3):
                fv.append((dx * Rs[k] + dy * Rs[3 + k] + dz * Rs[6 + k]) * inv_den)
            ninv = -inv_den
            for k in range(3):
                fv.append((dx * Rd[k] + dy * Rd[3 + k] + dz * Rd[6 + k]) * ninv)
            for a in range(3):
                for k in range(3):
                    fv.append(Rs[a] * Rd[k] + Rs[3 + a] * Rd[3 + k]
                              + Rs[6 + a] * Rd[6 + k])
            fv.append(zero)
            for f, v in enumerate(fv):
                plsc.store_scatter(fb, [rows, cols[f]], v)
            return carry

        lax.fori_loop(0, _GRP, grp, 0)

    fire_idx(isv0, idv0, s_idx0, 0)
    fire_idx(isv1, idv1, s_idx1, 1)
    wait_idx(isv0, idv0, s_idx0)
    fire_gathers(isv0, idv0, rs0, rd0, sgs0, sgd0)

    def pair(k, carry):
        a = 2 * k
        b = a + 1
        wait_idx(isv1, idv1, s_idx1)
        fire_gathers(isv1, idv1, rs1, rd1, sgs1, sgd1)
        wait_gathers(rs0, rd0, sgs0, sgd0)

        @pl.when(k < npair - 1)
        def _():
            fire_idx(isv0, idv0, s_idx0, a + 2)

        @pl.when(k >= 1)
        def _():
            wait_out(fb0, so0)

        compute_block(rs0, rd0, fb0)
        fire_out(fb0, so0, a)

        @pl.when(k < npair - 1)
        def _():
            wait_idx(isv0, idv0, s_idx0)
            fire_gathers(isv0, idv0, rs0, rd0, sgs0, sgd0)

        wait_gathers(rs1, rd1, sgs1, sgd1)

        @pl.when(k < npair - 1)
        def _():
            fire_idx(isv1, idv1, s_idx1, b + 2)

        @pl.when(k >= 1)
        def _():
            wait_out(fb1, so1)

        compute_block(rs1, rd1, fb1)
        fire_out(fb1, so1, b)
        return carry

    lax.fori_loop(0, npair, pair, 0)

    wait_out(fb0, so0)
    wait_out(fb1, so1)


_sc_feats = functools.partial(
    pl.kernel,
    out_type=jax.ShapeDtypeStruct((_N_EDGES // 4, 128), jnp.float32),
    mesh=plsc.VectorSubcoreMesh(core_axis_name="c", subcore_axis_name="s",
                                num_cores=_NC, num_subcores=_NS),
    scratch_types=[
        pltpu.VMEM((_BLK,), jnp.int32),
        pltpu.VMEM((_BLK,), jnp.int32),
        pltpu.VMEM((_BLK,), jnp.int32),
        pltpu.VMEM((_BLK,), jnp.int32),
        pltpu.VMEM((_BLK, 16), jnp.float32),
        pltpu.VMEM((_BLK, 16), jnp.float32),
        pltpu.VMEM((_BLK, 16), jnp.float32),
        pltpu.VMEM((_BLK, 16), jnp.float32),
        pltpu.VMEM((_BLK, _OUT_DIM), jnp.float32),
        pltpu.VMEM((_BLK, _OUT_DIM), jnp.float32),
        pltpu.SemaphoreType.DMA,
        pltpu.SemaphoreType.DMA,
        pltpu.SemaphoreType.DMA,
        pltpu.SemaphoreType.DMA,
        pltpu.SemaphoreType.DMA,
        pltpu.SemaphoreType.DMA,
        pltpu.SemaphoreType.DMA,
        pltpu.SemaphoreType.DMA,
    ],
    compiler_params=pltpu.CompilerParams(use_tc_tiling_on_sc=False,
                                         needs_layout_passes=False),
)(_sc_body)


def _mm_body(x_ref, w_ref, o_ref):
    y = jnp.dot(x_ref[...], w_ref[...], preferred_element_type=jnp.float32)
    o_ref[...] = jnp.concatenate(
        [y[:, 0:32], y[:, 32:64], y[:, 64:96], y[:, 96:128]], axis=0)


def _matmul(feats_packed, w_big):
    return pl.pallas_call(
        _mm_body,
        grid=(_NSB,),
        in_specs=[pl.BlockSpec((_TBS, 128), lambda i: (i, 0)),
                  pl.BlockSpec((128, 128), lambda i: (0, 0))],
        out_specs=pl.BlockSpec((_SBE, _OUT_DIM), lambda i: (i, 0)),
        out_shape=jax.ShapeDtypeStruct((_N_EDGES, _OUT_DIM), jnp.float32),
    )(feats_packed, w_big)


def kernel(frame_t, frame_R, edge_src, edge_dst, W):
    table = jnp.concatenate(
        [frame_t, frame_R.reshape(_N_NODES, 9),
         jnp.zeros((_N_NODES, 4), jnp.float32)], axis=1)
    w_big = jnp.zeros((128, 128), jnp.float32)
    for j in range(4):
        w_big = w_big.at[j * 32:j * 32 + 31, j * 32:(j + 1) * 32].set(W)
    feats_packed = _sc_feats(table, edge_src.astype(jnp.int32),
                             edge_dst.astype(jnp.int32))
    return _matmul(feats_packed, w_big)


# speedup vs baseline: 26.2355x; 1.4498x over previous
import functools

import jax
import jax.numpy as jnp
from jax import lax
from jax.experimental import pallas as pl
from jax.experimental.pallas import tpu as pltpu
from jax.experimental.pallas import tpu_sc as plsc

_N_NODES = 100000
_N_EDGES = 1600000
_N_BASIS = 16
_OUT_DIM = 32
_D_MAX = 20.0

_NC = 2
_NS = 16
_NW = _NC * _NS
_L = 16

_BLK = 400
_SUB = 80
_NSUB = _BLK // _SUB
_GRP = _BLK // _L

_SBE = 6400
_NSB = _N_EDGES // _SBE
_TBS = _SBE // 4
_SB8 = _NSB - 7 * _NW


def _rsqrt(a):
    i = plsc.bitcast(a, jnp.int32)
    y = plsc.bitcast(jnp.int32(0x5F3759DF) - (i >> 1), jnp.float32)
    h = 0.5 * a
    for _ in range(2):
        y = y * (1.5 - h * y * y)
    return y


def _sc_body(table, esrc, edst, feats,
             isv0, idv0, isv1, idv1,
             rs0, rd0, rs1, rd1,
             fb0, fb1,
             s_idx0, s_idx1, sgs0, sgd0, sgs1, sgd1, so0, so1):
    cid = lax.axis_index("c")
    sid = lax.axis_index("s")
    wid = sid * _NC + cid
    sb_start = jnp.where(wid < _SB8, 8 * wid, 7 * wid + _SB8)
    nblk = jnp.where(wid < _SB8, 8 * _SBE // _BLK, 7 * _SBE // _BLK)
    npair = nblk // 2
    wbase = sb_start * _SBE
    ii = lax.iota(jnp.int32, _L)
    centers = [jnp.float32(_D_MAX * b / (_N_BASIS - 1)) for b in range(_N_BASIS)]
    zero = jnp.zeros((_L,), jnp.float32)
    cols = [jnp.full((_L,), f, jnp.int32) for f in range(_OUT_DIM)]

    def fire_idx(iref, dref, sem, blk):
        ebase = wbase + blk * _BLK
        pltpu.async_copy(esrc.at[pl.ds(ebase, _BLK)], iref, sem)
        pltpu.async_copy(edst.at[pl.ds(ebase, _BLK)], dref, sem)

    def wait_idx(iref, dref, sem):
        pltpu.make_async_copy(esrc.at[pl.ds(0, _BLK)], iref, sem).wait()
        pltpu.make_async_copy(esrc.at[pl.ds(0, _BLK)], dref, sem).wait()

    def fire_gathers(iref, dref, rs, rd, sgs, sgd):
        for k in range(_NSUB):
            sl = pl.ds(k * _SUB, _SUB)
            pltpu.async_copy(table.at[iref.at[sl]], rs.at[sl], sgs)
            pltpu.async_copy(table.at[dref.at[sl]], rd.at[sl], sgd)

    def wait_gathers(rs, rd, sgs, sgd):
        pltpu.make_async_copy(table.at[pl.ds(0, _BLK)], rs, sgs).wait()
        pltpu.make_async_copy(table.at[pl.ds(0, _BLK)], rd, sgd).wait()

    def fire_out(fb, sem, blk):
        eb = wbase + blk * _BLK
        sb = eb // _SBE
        off = eb % _SBE
        jb = off // _TBS
        r0 = pl.multiple_of(sb * _TBS + off % _TBS, _BLK)
        pltpu.async_copy(
            fb, feats.at[pl.ds(r0, _BLK),
                         pl.ds(pl.multiple_of(jb * 32, 32), 32)], sem)

    def wait_out(fb, sem):
        pltpu.make_async_copy(
            fb, feats.at[pl.ds(0, _BLK), pl.ds(0, 32)], sem).wait()

    def compute_block(rs, rd, fb):
        def grp(g, carry):
            rows = g * _L + ii
            s = [plsc.load_gather(rs, [rows, cols[f]]) for f in range(12)]
            d = [plsc.load_gather(rd, [rows, cols[f]]) for f in range(12)]
            dx = d[0] - s[0]
            dy = d[1] - s[1]
            dz = d[2] - s[2]
            d2 = dx * dx + dy * dy + dz * dz
            dist = d2 * _rsqrt(jnp.maximum(d2, jnp.float32(1e-20)))
            inv_den = _rsqrt(d2 + 1.0)
            Rs = s[3:]
            Rd = d[3:]
            fv = []
            for c in centers:
                t = dist - c
                fv.append(jnp.exp(-(t * t)))
            for k in range(3):
                fv.append((dx * Rs[k] + dy * Rs[3 + k] + dz * Rs[6 + k]) * inv_den)
            ninv = -inv_den
            for k in range(3):
                fv.append((dx * Rd[k] + dy * Rd[3 + k] + dz * Rd[6 + k]) * ninv)
            for a in range(3):
                for k in range(3):
                    fv.append(Rs[a] * Rd[k] + Rs[3 + a] * Rd[3 + k]
                              + Rs[6 + a] * Rd[6 + k])
            fv.append(zero)
            for f, v in enumerate(fv):
                plsc.store_scatter(fb, [rows, cols[f]], v)
            return carry

        lax.fori_loop(0, _GRP, grp, 0)

    fire_idx(isv0, idv0, s_idx0, 0)
    fire_idx(isv1, idv1, s_idx1, 1)
    wait_idx(isv0, idv0, s_idx0)
    fire_gathers(isv0, idv0, rs0, rd0, sgs0, sgd0)

    def pair(k, carry):
        a = 2 * k
        b = a + 1
        wait_idx(isv1, idv1, s_idx1)
        fire_gathers(isv1, idv1, rs1, rd1, sgs1, sgd1)
        wait_gathers(rs0, rd0, sgs0, sgd0)

        @pl.when(k < npair - 1)
        def _():
            fire_idx(isv0, idv0, s_idx0, a + 2)

        @pl.when(k >= 1)
        def _():
            wait_out(fb0, so0)

        compute_block(rs0, rd0, fb0)
        fire_out(fb0, so0, a)

        @pl.when(k < npair - 1)
        def _():
            wait_idx(isv0, idv0, s_idx0)
            fire_gathers(isv0, idv0, rs0, rd0, sgs0, sgd0)

        wait_gathers(rs1, rd1, sgs1, sgd1)

        @pl.when(k < npair - 1)
        def _():
            fire_idx(isv1, idv1, s_idx1, b + 2)

        @pl.when(k >= 1)
        def _():
            wait_out(fb1, so1)

        compute_block(rs1, rd1, fb1)
        fire_out(fb1, so1, b)
        return carry

    lax.fori_loop(0, npair, pair, 0)

    wait_out(fb0, so0)
    wait_out(fb1, so1)


_sc_feats = functools.partial(
    pl.kernel,
    out_type=jax.ShapeDtypeStruct((_N_EDGES // 4, 128), jnp.float32),
    mesh=plsc.VectorSubcoreMesh(core_axis_name="c", subcore_axis_name="s",
                                num_cores=_NC, num_subcores=_NS),
    scratch_types=[
        pltpu.VMEM((_BLK,), jnp.int32),
        pltpu.VMEM((_BLK,), jnp.int32),
        pltpu.VMEM((_BLK,), jnp.int32),
        pltpu.VMEM((_BLK,), jnp.int32),
        pltpu.VMEM((_BLK, 16), jnp.float32),
        pltpu.VMEM((_BLK, 16), jnp.float32),
        pltpu.VMEM((_BLK, 16), jnp.float32),
        pltpu.VMEM((_BLK, 16), jnp.float32),
        pltpu.VMEM((_BLK, _OUT_DIM), jnp.float32),
        pltpu.VMEM((_BLK, _OUT_DIM), jnp.float32),
        pltpu.SemaphoreType.DMA,
        pltpu.SemaphoreType.DMA,
        pltpu.SemaphoreType.DMA,
        pltpu.SemaphoreType.DMA,
        pltpu.SemaphoreType.DMA,
        pltpu.SemaphoreType.DMA,
        pltpu.SemaphoreType.DMA,
        pltpu.SemaphoreType.DMA,
    ],
    compiler_params=pltpu.CompilerParams(use_tc_tiling_on_sc=False,
                                         needs_layout_passes=False),
)(_sc_body)


def _mm_body(w_ref, x_ref, o_ref):
    zt = lax.dot_general(w_ref[...], x_ref[...], (((0,), (1,)), ((), ())),
                         preferred_element_type=jnp.float32)
    o_ref[...] = jnp.concatenate(
        [zt[0:32, :], zt[32:64, :], zt[64:96, :], zt[96:128, :]], axis=1)


def _matmul(w_big, feats_packed):
    return pl.pallas_call(
        _mm_body,
        grid=(_NSB,),
        in_specs=[pl.BlockSpec((128, 128), lambda i: (0, 0)),
                  pl.BlockSpec((_TBS, 128), lambda i: (i, 0))],
        out_specs=pl.BlockSpec((_OUT_DIM, _SBE), lambda i: (0, i)),
        out_shape=jax.ShapeDtypeStruct((_OUT_DIM, _N_EDGES), jnp.float32),
    )(w_big, feats_packed)


def kernel(frame_t, frame_R, edge_src, edge_dst, W):
    table = jnp.concatenate(
        [frame_t, frame_R.reshape(_N_NODES, 9),
         jnp.zeros((_N_NODES, 4), jnp.float32)], axis=1)
    w_big = jnp.zeros((128, 128), jnp.float32)
    for j in range(4):
        w_big = w_big.at[j * 32:j * 32 + 31, j * 32:(j + 1) * 32].set(W)
    feats_packed = _sc_feats(table, edge_src.astype(jnp.int32),
                             edge_dst.astype(jnp.int32))
    out_t = _matmul(w_big, feats_packed)
    return out_t.T


# speedup vs baseline: 27.9600x; 1.0657x over previous
import functools
import math

import jax
import jax.numpy as jnp
from jax import lax
from jax.experimental import pallas as pl
from jax.experimental.pallas import tpu as pltpu
from jax.experimental.pallas import tpu_sc as plsc

_N_NODES = 100000
_N_EDGES = 1600000
_N_BASIS = 16
_OUT_DIM = 32
_D_MAX = 20.0

_NC = 2
_NS = 16
_NW = _NC * _NS
_L = 16

_BLK = 400
_SUB = 80
_NSUB = _BLK // _SUB
_GRP = _BLK // _L

_SBE = 6400
_NSB = _N_EDGES // _SBE
_TBS = _SBE // 4
_SB8 = _NSB - 7 * _NW


def _rsqrt(a):
    i = plsc.bitcast(a, jnp.int32)
    y = plsc.bitcast(jnp.int32(0x5F3759DF) - (i >> 1), jnp.float32)
    h = 0.5 * a
    for _ in range(2):
        y = y * (1.5 - (h * y) * y)
    return y


def _sc_body(table, esrc, edst, feats,
             isv0, idv0, isv1, idv1,
             rs0, rd0, rs1, rd1,
             fb0, fb1,
             s_idx0, s_idx1, sgs0, sgd0, sgs1, sgd1, so0, so1):
    cid = lax.axis_index("c")
    sid = lax.axis_index("s")
    wid = sid * _NC + cid
    sb_start = jnp.where(wid < _SB8, 8 * wid, 7 * wid + _SB8)
    nblk = jnp.where(wid < _SB8, 8 * _SBE // _BLK, 7 * _SBE // _BLK)
    npair = nblk // 2
    wbase = sb_start * _SBE
    ii = lax.iota(jnp.int32, _L)
    zero = jnp.zeros((_L,), jnp.float32)
    cols = [jnp.full((_L,), f, jnp.int32) for f in range(_OUT_DIM)]
    beta = _D_MAX / (_N_BASIS - 1)
    kconst = [jnp.float32(math.exp(-beta * beta * (2 * b + 1)))
              for b in range(_N_BASIS - 1)]

    def fire_idx(iref, dref, sem, blk):
        ebase = wbase + blk * _BLK
        pltpu.async_copy(esrc.at[pl.ds(ebase, _BLK)], iref, sem)
        pltpu.async_copy(edst.at[pl.ds(ebase, _BLK)], dref, sem)

    def wait_idx(iref, dref, sem):
        pltpu.make_async_copy(esrc.at[pl.ds(0, _BLK)], iref, sem).wait()
        pltpu.make_async_copy(esrc.at[pl.ds(0, _BLK)], dref, sem).wait()

    def fire_gathers(iref, dref, rs, rd, sgs, sgd):
        for k in range(_NSUB):
            sl = pl.ds(k * _SUB, _SUB)
            pltpu.async_copy(table.at[iref.at[sl]], rs.at[sl], sgs)
            pltpu.async_copy(table.at[dref.at[sl]], rd.at[sl], sgd)

    def wait_gathers(rs, rd, sgs, sgd):
        pltpu.make_async_copy(table.at[pl.ds(0, _BLK)], rs, sgs).wait()
        pltpu.make_async_copy(table.at[pl.ds(0, _BLK)], rd, sgd).wait()

    def fire_out(fb, sem, blk):
        eb = wbase + blk * _BLK
        sb = eb // _SBE
        off = eb % _SBE
        jb = off // _TBS
        r0 = pl.multiple_of(sb * _TBS + off % _TBS, _BLK)
        pltpu.async_copy(
            fb, feats.at[pl.ds(r0, _BLK),
                         pl.ds(pl.multiple_of(jb * 32, 32), 32)], sem)

    def wait_out(fb, sem):
        pltpu.make_async_copy(
            fb, feats.at[pl.ds(0, _BLK), pl.ds(0, 32)], sem).wait()

    def compute_block(rs, rd, fb):
        def grp(g, carry):
            rows = g * _L + ii
            s = [plsc.load_gather(rs, [rows, cols[f]]) for f in range(12)]
            d = [plsc.load_gather(rd, [rows, cols[f]]) for f in range(12)]
            dx = d[0] - s[0]
            dy = d[1] - s[1]
            dz = d[2] - s[2]
            d2 = dx * dx + dy * dy + dz * dz
            dist = d2 * _rsqrt(d2)
            inv_den = _rsqrt(d2 + 1.0)
            Rs = s[3:]
            Rd = d[3:]
            fv = []
            r = jnp.exp(-d2)
            u = jnp.exp((2.0 * beta) * dist)
            fv.append(r)
            for kc in kconst:
                r = r * (u * kc)
                fv.append(r)
            for k in range(3):
                fv.append((dx * Rs[k] + dy * Rs[3 + k] + dz * Rs[6 + k]) * inv_den)
            ninv = -inv_den
            for k in range(3):
                fv.append((dx * Rd[k] + dy * Rd[3 + k] + dz * Rd[6 + k]) * ninv)
            for a in range(3):
                for k in range(3):
                    fv.append(Rs[a] * Rd[k] + Rs[3 + a] * Rd[3 + k]
                              + Rs[6 + a] * Rd[6 + k])
            fv.append(zero)
            for f, v in enumerate(fv):
                plsc.store_scatter(fb, [rows, cols[f]], v)
            return carry

        lax.fori_loop(0, _GRP, grp, 0)

    fire_idx(isv0, idv0, s_idx0, 0)
    fire_idx(isv1, idv1, s_idx1, 1)
    wait_idx(isv0, idv0, s_idx0)
    fire_gathers(isv0, idv0, rs0, rd0, sgs0, sgd0)

    def pair(k, carry):
        a = 2 * k
        b = a + 1
        wait_idx(isv1, idv1, s_idx1)
        fire_gathers(isv1, idv1, rs1, rd1, sgs1, sgd1)
        wait_gathers(rs0, rd0, sgs0, sgd0)

        @pl.when(k < npair - 1)
        def _():
            fire_idx(isv0, idv0, s_idx0, a + 2)

        @pl.when(k >= 1)
        def _():
            wait_out(fb0, so0)

        compute_block(rs0, rd0, fb0)
        fire_out(fb0, so0, a)

        @pl.when(k < npair - 1)
        def _():
            wait_idx(isv0, idv0, s_idx0)
            fire_gathers(isv0, idv0, rs0, rd0, sgs0, sgd0)

        wait_gathers(rs1, rd1, sgs1, sgd1)

        @pl.when(k < npair - 1)
        def _():
            fire_idx(isv1, idv1, s_idx1, b + 2)

        @pl.when(k >= 1)
        def _():
            wait_out(fb1, so1)

        compute_block(rs1, rd1, fb1)
        fire_out(fb1, so1, b)
        return carry

    lax.fori_loop(0, npair, pair, 0)

    wait_out(fb0, so0)
    wait_out(fb1, so1)


_sc_feats = functools.partial(
    pl.kernel,
    out_type=jax.ShapeDtypeStruct((_N_EDGES // 4, 128), jnp.float32),
    mesh=plsc.VectorSubcoreMesh(core_axis_name="c", subcore_axis_name="s",
                                num_cores=_NC, num_subcores=_NS),
    scratch_types=[
        pltpu.VMEM((_BLK,), jnp.int32),
        pltpu.VMEM((_BLK,), jnp.int32),
        pltpu.VMEM((_BLK,), jnp.int32),
        pltpu.VMEM((_BLK,), jnp.int32),
        pltpu.VMEM((_BLK, 16), jnp.float32),
        pltpu.VMEM((_BLK, 16), jnp.float32),
        pltpu.VMEM((_BLK, 16), jnp.float32),
        pltpu.VMEM((_BLK, 16), jnp.float32),
        pltpu.VMEM((_BLK, _OUT_DIM), jnp.float32),
        pltpu.VMEM((_BLK, _OUT_DIM), jnp.float32),
        pltpu.SemaphoreType.DMA,
        pltpu.SemaphoreType.DMA,
        pltpu.SemaphoreType.DMA,
        pltpu.SemaphoreType.DMA,
        pltpu.SemaphoreType.DMA,
        pltpu.SemaphoreType.DMA,
        pltpu.SemaphoreType.DMA,
        pltpu.SemaphoreType.DMA,
    ],
    compiler_params=pltpu.CompilerParams(use_tc_tiling_on_sc=False,
                                         needs_layout_passes=False),
)(_sc_body)


def _mm_body(w_ref, x_ref, o_ref):
    zt = lax.dot_general(w_ref[...], x_ref[...], (((0,), (1,)), ((), ())),
                         preferred_element_type=jnp.float32)
    o_ref[...] = jnp.concatenate(
        [zt[0:32, :], zt[32:64, :], zt[64:96, :], zt[96:128, :]], axis=1)


def _matmul(w_big, feats_packed):
    return pl.pallas_call(
        _mm_body,
        grid=(_NSB,),
        in_specs=[pl.BlockSpec((128, 128), lambda i: (0, 0)),
                  pl.BlockSpec((_TBS, 128), lambda i: (i, 0))],
        out_specs=pl.BlockSpec((_OUT_DIM, _SBE), lambda i: (0, i)),
        out_shape=jax.ShapeDtypeStruct((_OUT_DIM, _N_EDGES), jnp.float32),
    )(w_big, feats_packed)


def kernel(frame_t, frame_R, edge_src, edge_dst, W):
    table = jnp.concatenate(
        [frame_t, frame_R.reshape(_N_NODES, 9),
         jnp.zeros((_N_NODES, 4), jnp.float32)], axis=1)
    w_big = jnp.zeros((128, 128), jnp.float32)
    for j in range(4):
        w_big = w_big.at[j * 32:j * 32 + 31, j * 32:(j + 1) * 32].set(W)
    feats_packed = _sc_feats(table, edge_src.astype(jnp.int32),
                             edge_dst.astype(jnp.int32))
    out_t = _matmul(w_big, feats_packed)
    return out_t.T


# speedup vs baseline: 30.6213x; 1.0952x over previous
import functools
import math

import jax
import jax.numpy as jnp
from jax import lax
from jax.experimental import pallas as pl
from jax.experimental.pallas import tpu as pltpu
from jax.experimental.pallas import tpu_sc as plsc

_N_NODES = 100000
_N_EDGES = 1600000
_N_BASIS = 16
_OUT_DIM = 32
_D_MAX = 20.0

_NC = 2
_NS = 16
_NW = _NC * _NS
_L = 16

_BLK = 400
_SUB = 80
_NSUB = _BLK // _SUB
_GRP = _BLK // _L

_SBE = 6400
_NSB = _N_EDGES // _SBE
_TBS = _SBE // 4
_EH = _N_EDGES // 2
_NSBH = _EH // _SBE


def _rsqrt(a):
    i = plsc.bitcast(a, jnp.int32)
    y = plsc.bitcast(jnp.int32(0x5F3759DF) - (i >> 1), jnp.float32)
    h = 0.5 * a
    for _ in range(2):
        y = y * (1.5 - (h * y) * y)
    return y


def _sc_body(table, esrc, edst, feats,
             isv0, idv0, isv1, idv1,
             rs0, rd0, rs1, rd1,
             fb0, fb1,
             s_idx0, s_idx1, sgs0, sgd0, sgs1, sgd1, so0, so1):
    n_sb = _NSBH
    q, rm = divmod(n_sb, _NW)
    cid = lax.axis_index("c")
    sid = lax.axis_index("s")
    wid = sid * _NC + cid
    sb_start = jnp.where(wid < rm, (q + 1) * wid, q * wid + rm)
    nblk = jnp.where(wid < rm, (q + 1) * _SBE // _BLK, q * _SBE // _BLK)
    npair = nblk // 2
    wbase = sb_start * _SBE
    ii = lax.iota(jnp.int32, _L)
    zero = jnp.zeros((_L,), jnp.float32)
    cols = [jnp.full((_L,), f, jnp.int32) for f in range(_OUT_DIM)]
    beta = _D_MAX / (_N_BASIS - 1)
    kconst = [jnp.float32(math.exp(-beta * beta * (2 * b + 1)))
              for b in range(_N_BASIS - 1)]

    def fire_idx(iref, dref, sem, blk):
        ebase = wbase + blk * _BLK
        pltpu.async_copy(esrc.at[pl.ds(ebase, _BLK)], iref, sem)
        pltpu.async_copy(edst.at[pl.ds(ebase, _BLK)], dref, sem)

    def wait_idx(iref, dref, sem):
        pltpu.make_async_copy(esrc.at[pl.ds(0, _BLK)], iref, sem).wait()
        pltpu.make_async_copy(esrc.at[pl.ds(0, _BLK)], dref, sem).wait()

    def fire_gathers(iref, dref, rs, rd, sgs, sgd):
        for k in range(_NSUB):
            sl = pl.ds(k * _SUB, _SUB)
            pltpu.async_copy(table.at[iref.at[sl]], rs.at[sl], sgs)
            pltpu.async_copy(table.at[dref.at[sl]], rd.at[sl], sgd)

    def wait_gathers(rs, rd, sgs, sgd):
        pltpu.make_async_copy(table.at[pl.ds(0, _BLK)], rs, sgs).wait()
        pltpu.make_async_copy(table.at[pl.ds(0, _BLK)], rd, sgd).wait()

    def fire_out(fb, sem, blk):
        eb = wbase + blk * _BLK
        sb = eb // _SBE
        off = eb % _SBE
        jb = off // _TBS
        r0 = pl.multiple_of(sb * _TBS + off % _TBS, _BLK)
        pltpu.async_copy(
            fb, feats.at[pl.ds(r0, _BLK),
                         pl.ds(pl.multiple_of(jb * 32, 32), 32)], sem)

    def wait_out(fb, sem):
        pltpu.make_async_copy(
            fb, feats.at[pl.ds(0, _BLK), pl.ds(0, 32)], sem).wait()

    def compute_block(rs, rd, fb):
        def grp(g, carry):
            rows = g * _L + ii
            s = [plsc.load_gather(rs, [rows, cols[f]]) for f in range(12)]
            d = [plsc.load_gather(rd, [rows, cols[f]]) for f in range(12)]
            dx = d[0] - s[0]
            dy = d[1] - s[1]
            dz = d[2] - s[2]
            d2 = dx * dx + dy * dy + dz * dz
            dist = d2 * _rsqrt(d2)
            inv_den = _rsqrt(d2 + 1.0)
            Rs = s[3:]
            Rd = d[3:]
            fv = []
            r = jnp.exp(-d2)
            u = jnp.exp((2.0 * beta) * dist)
            fv.append(r)
            for kc in kconst:
                r = r * (u * kc)
                fv.append(r)
            for k in range(3):
                fv.append((dx * Rs[k] + dy * Rs[3 + k] + dz * Rs[6 + k]) * inv_den)
            ninv = -inv_den
            for k in range(3):
                fv.append((dx * Rd[k] + dy * Rd[3 + k] + dz * Rd[6 + k]) * ninv)
            for a in range(3):
                for k in range(3):
                    fv.append(Rs[a] * Rd[k] + Rs[3 + a] * Rd[3 + k]
                              + Rs[6 + a] * Rd[6 + k])
            fv.append(zero)
            for f, v in enumerate(fv):
                plsc.store_scatter(fb, [rows, cols[f]], v)
            return carry

        lax.fori_loop(0, _GRP, grp, 0)

    fire_idx(isv0, idv0, s_idx0, 0)
    fire_idx(isv1, idv1, s_idx1, 1)
    wait_idx(isv0, idv0, s_idx0)
    fire_gathers(isv0, idv0, rs0, rd0, sgs0, sgd0)

    def pair(k, carry):
        a = 2 * k
        b = a + 1
        wait_idx(isv1, idv1, s_idx1)
        fire_gathers(isv1, idv1, rs1, rd1, sgs1, sgd1)
        wait_gathers(rs0, rd0, sgs0, sgd0)

        @pl.when(k < npair - 1)
        def _():
            fire_idx(isv0, idv0, s_idx0, a + 2)

        @pl.when(k >= 1)
        def _():
            wait_out(fb0, so0)

        compute_block(rs0, rd0, fb0)
        fire_out(fb0, so0, a)

        @pl.when(k < npair - 1)
        def _():
            wait_idx(isv0, idv0, s_idx0)
            fire_gathers(isv0, idv0, rs0, rd0, sgs0, sgd0)

        wait_gathers(rs1, rd1, sgs1, sgd1)

        @pl.when(k < npair - 1)
        def _():
            fire_idx(isv1, idv1, s_idx1, b + 2)

        @pl.when(k >= 1)
        def _():
            wait_out(fb1, so1)

        compute_block(rs1, rd1, fb1)
        fire_out(fb1, so1, b)
        return carry

    lax.fori_loop(0, npair, pair, 0)

    wait_out(fb0, so0)
    wait_out(fb1, so1)


_sc_feats = functools.partial(
    pl.kernel,
    out_type=jax.ShapeDtypeStruct((_EH // 4, 128), jnp.float32),
    mesh=plsc.VectorSubcoreMesh(core_axis_name="c", subcore_axis_name="s",
                                num_cores=_NC, num_subcores=_NS),
    scratch_types=[
        pltpu.VMEM((_BLK,), jnp.int32),
        pltpu.VMEM((_BLK,), jnp.int32),
        pltpu.VMEM((_BLK,), jnp.int32),
        pltpu.VMEM((_BLK,), jnp.int32),
        pltpu.VMEM((_BLK, 16), jnp.float32),
        pltpu.VMEM((_BLK, 16), jnp.float32),
        pltpu.VMEM((_BLK, 16), jnp.float32),
        pltpu.VMEM((_BLK, 16), jnp.float32),
        pltpu.VMEM((_BLK, _OUT_DIM), jnp.float32),
        pltpu.VMEM((_BLK, _OUT_DIM), jnp.float32),
        pltpu.SemaphoreType.DMA,
        pltpu.SemaphoreType.DMA,
        pltpu.SemaphoreType.DMA,
        pltpu.SemaphoreType.DMA,
        pltpu.SemaphoreType.DMA,
        pltpu.SemaphoreType.DMA,
        pltpu.SemaphoreType.DMA,
        pltpu.SemaphoreType.DMA,
    ],
    compiler_params=pltpu.CompilerParams(use_tc_tiling_on_sc=False,
                                         needs_layout_passes=False),
)(_sc_body)


def _mm_body(w_ref, x_ref, o_ref):
    zt = lax.dot_general(w_ref[...], x_ref[...], (((0,), (1,)), ((), ())),
                         preferred_element_type=jnp.float32)
    o_ref[...] = jnp.concatenate(
        [zt[0:32, :], zt[32:64, :], zt[64:96, :], zt[96:128, :]], axis=1)


def _mm_body_alias(w_ref, x_ref, prev_ref, o_ref):
    del prev_ref
    _mm_body(w_ref, x_ref, o_ref)


def _matmul_first(w_big, feats_packed):
    return pl.pallas_call(
        _mm_body,
        grid=(_NSBH,),
        in_specs=[pl.BlockSpec((128, 128), lambda i: (0, 0)),
                  pl.BlockSpec((_TBS, 128), lambda i: (i, 0))],
        out_specs=pl.BlockSpec((_OUT_DIM, _SBE), lambda i: (0, i)),
        out_shape=jax.ShapeDtypeStruct((_OUT_DIM, _N_EDGES), jnp.float32),
    )(w_big, feats_packed)


def _matmul_second(w_big, feats_packed, prev):
    return pl.pallas_call(
        _mm_body_alias,
        grid=(_NSBH,),
        in_specs=[pl.BlockSpec((128, 128), lambda i: (0, 0)),
                  pl.BlockSpec((_TBS, 128), lambda i: (i, 0)),
                  pl.BlockSpec(memory_space=pltpu.HBM)],
        out_specs=pl.BlockSpec((_OUT_DIM, _SBE), lambda i: (0, i + _NSBH)),
        out_shape=jax.ShapeDtypeStruct((_OUT_DIM, _N_EDGES), jnp.float32),
        input_output_aliases={2: 0},
    )(w_big, feats_packed, prev)


def kernel(frame_t, frame_R, edge_src, edge_dst, W):
    table = jnp.concatenate(
        [frame_t, frame_R.reshape(_N_NODES, 9),
         jnp.zeros((_N_NODES, 4), jnp.float32)], axis=1)
    w_big = jnp.zeros((128, 128), jnp.float32)
    for j in range(4):
        w_big = w_big.at[j * 32:j * 32 + 31, j * 32:(j + 1) * 32].set(W)
    esrc = edge_src.astype(jnp.int32)
    edst = edge_dst.astype(jnp.int32)
    f1 = _sc_feats(table, esrc[:_EH], edst[:_EH])
    f2 = _sc_feats(table, esrc[_EH:], edst[_EH:])
    o1 = _matmul_first(w_big, f1)
    out_t = _matmul_second(w_big, f2, o1)
    return out_t.T


# speedup vs baseline: 44.5395x; 1.4545x over previous
import functools
import math

import jax
import jax.numpy as jnp
from jax import lax
from jax.experimental import pallas as pl
from jax.experimental.pallas import tpu as pltpu
from jax.experimental.pallas import tpu_sc as plsc

_N_NODES = 100000
_N_EDGES = 1600000
_N_BASIS = 16
_OUT_DIM = 32
_D_MAX = 20.0

_NC = 2
_NS = 16
_NW = _NC * _NS
_L = 16

_BLK = 400
_SUB = 80
_NSUB = _BLK // _SUB
_GRP = _BLK // _L

_SBE = 6400
_NSB = _N_EDGES // _SBE
_TBS = _SBE // 4
_EH = _N_EDGES // 2
_NSBH = _EH // _SBE


def _rsqrt(a):
    i = plsc.bitcast(a, jnp.int32)
    y = plsc.bitcast(jnp.int32(0x5F3759DF) - (i >> 1), jnp.float32)
    h = 0.5 * a
    for _ in range(2):
        y = y * (1.5 - (h * y) * y)
    return y


def _sc_body(table, esrc, edst, feats,
             isv0, idv0, isv1, idv1,
             rs0, rd0, rs1, rd1,
             fb0, fb1,
             s_idx0, s_idx1, sgs0, sgd0, sgs1, sgd1, so0, so1):
    n_sb = _NSBH
    q, rm = divmod(n_sb, _NW)
    cid = lax.axis_index("c")
    sid = lax.axis_index("s")
    wid = sid * _NC + cid
    sb_start = jnp.where(wid < rm, (q + 1) * wid, q * wid + rm)
    nblk = jnp.where(wid < rm, (q + 1) * _SBE // _BLK, q * _SBE // _BLK)
    npair = nblk // 2
    wbase = sb_start * _SBE
    ii = lax.iota(jnp.int32, _L)
    zero = jnp.zeros((_L,), jnp.float32)
    cols = [jnp.full((_L,), f, jnp.int32) for f in range(_OUT_DIM)]
    beta = _D_MAX / (_N_BASIS - 1)
    kconst = [jnp.float32(math.exp(-beta * beta * (2 * b + 1)))
              for b in range(_N_BASIS - 1)]

    def fire_idx(iref, dref, sem, blk):
        ebase = wbase + blk * _BLK
        pltpu.async_copy(esrc.at[pl.ds(ebase, _BLK)], iref, sem)
        pltpu.async_copy(edst.at[pl.ds(ebase, _BLK)], dref, sem)

    def wait_idx(iref, dref, sem):
        pltpu.make_async_copy(esrc.at[pl.ds(0, _BLK)], iref, sem).wait()
        pltpu.make_async_copy(esrc.at[pl.ds(0, _BLK)], dref, sem).wait()

    def fire_gathers(iref, dref, rs, rd, sgs, sgd):
        for k in range(_NSUB):
            sl = pl.ds(k * _SUB, _SUB)
            pltpu.async_copy(table.at[iref.at[sl]], rs.at[sl], sgs)
            pltpu.async_copy(table.at[dref.at[sl]], rd.at[sl], sgd)

    def wait_gathers(rs, rd, sgs, sgd):
        pltpu.make_async_copy(table.at[pl.ds(0, _BLK)], rs, sgs).wait()
        pltpu.make_async_copy(table.at[pl.ds(0, _BLK)], rd, sgd).wait()

    def fire_out(fb, sem, blk):
        eb = wbase + blk * _BLK
        sb = eb // _SBE
        off = eb % _SBE
        jb = off // _TBS
        r0 = pl.multiple_of(sb * _TBS + off % _TBS, _BLK)
        pltpu.async_copy(
            fb.at[pl.ds(0, _BLK), pl.ds(0, 32)],
            feats.at[pl.ds(r0, _BLK),
                     pl.ds(pl.multiple_of(jb * 32, 32), 32)], sem)

    def wait_out(fb, sem):
        pltpu.make_async_copy(
            fb.at[pl.ds(0, _BLK), pl.ds(0, 32)],
            feats.at[pl.ds(0, _BLK), pl.ds(0, 32)], sem).wait()

    def compute_block(rs, rd, fb):
        def grp(g, carry):
            rows = g * _L + ii
            s = [plsc.load_gather(rs, [rows, cols[f]]) for f in range(12)]
            d = [plsc.load_gather(rd, [rows, cols[f]]) for f in range(12)]
            dx = d[0] - s[0]
            dy = d[1] - s[1]
            dz = d[2] - s[2]
            d2 = dx * dx + dy * dy + dz * dz
            dist = d2 * _rsqrt(d2)
            inv_den = _rsqrt(d2 + 1.0)
            Rs = s[3:]
            Rd = d[3:]
            fv = []
            r = jnp.exp(-d2)
            u = jnp.exp((2.0 * beta) * dist)
            fv.append(r)
            for kc in kconst:
                r = r * (u * kc)
                fv.append(r)
            for k in range(3):
                fv.append((dx * Rs[k] + dy * Rs[3 + k] + dz * Rs[6 + k]) * inv_den)
            ninv = -inv_den
            for k in range(3):
                fv.append((dx * Rd[k] + dy * Rd[3 + k] + dz * Rd[6 + k]) * ninv)
            for a in range(3):
                for k in range(3):
                    fv.append(Rs[a] * Rd[k] + Rs[3 + a] * Rd[3 + k]
                              + Rs[6 + a] * Rd[6 + k])
            fv.append(zero)
            for f, v in enumerate(fv):
                plsc.store_scatter(fb, [rows, cols[f]], v)
            return carry

        lax.fori_loop(0, _GRP, grp, 0)

    fire_idx(isv0, idv0, s_idx0, 0)
    fire_idx(isv1, idv1, s_idx1, 1)
    wait_idx(isv0, idv0, s_idx0)
    fire_gathers(isv0, idv0, rs0, rd0, sgs0, sgd0)

    def pair(k, carry):
        a = 2 * k
        b = a + 1
        wait_idx(isv1, idv1, s_idx1)
        fire_gathers(isv1, idv1, rs1, rd1, sgs1, sgd1)
        wait_gathers(rs0, rd0, sgs0, sgd0)

        @pl.when(k < npair - 1)
        def _():
            fire_idx(isv0, idv0, s_idx0, a + 2)

        @pl.when(k >= 1)
        def _():
            wait_out(fb0, so0)

        compute_block(rs0, rd0, fb0)
        fire_out(fb0, so0, a)

        @pl.when(k < npair - 1)
        def _():
            wait_idx(isv0, idv0, s_idx0)
            fire_gathers(isv0, idv0, rs0, rd0, sgs0, sgd0)

        wait_gathers(rs1, rd1, sgs1, sgd1)

        @pl.when(k < npair - 1)
        def _():
            fire_idx(isv1, idv1, s_idx1, b + 2)

        @pl.when(k >= 1)
        def _():
            wait_out(fb1, so1)

        compute_block(rs1, rd1, fb1)
        fire_out(fb1, so1, b)
        return carry

    lax.fori_loop(0, npair, pair, 0)

    wait_out(fb0, so0)
    wait_out(fb1, so1)


_sc_feats = functools.partial(
    pl.kernel,
    out_type=jax.ShapeDtypeStruct((_EH // 4, 128), jnp.float32),
    mesh=plsc.VectorSubcoreMesh(core_axis_name="c", subcore_axis_name="s",
                                num_cores=_NC, num_subcores=_NS),
    scratch_types=[
        pltpu.VMEM((_BLK,), jnp.int32),
        pltpu.VMEM((_BLK,), jnp.int32),
        pltpu.VMEM((_BLK,), jnp.int32),
        pltpu.VMEM((_BLK,), jnp.int32),
        pltpu.VMEM((_BLK, 16), jnp.float32),
        pltpu.VMEM((_BLK, 16), jnp.float32),
        pltpu.VMEM((_BLK, 16), jnp.float32),
        pltpu.VMEM((_BLK, 16), jnp.float32),
        pltpu.VMEM((_BLK, 33), jnp.float32),
        pltpu.VMEM((_BLK, 33), jnp.float32),
        pltpu.SemaphoreType.DMA,
        pltpu.SemaphoreType.DMA,
        pltpu.SemaphoreType.DMA,
        pltpu.SemaphoreType.DMA,
        pltpu.SemaphoreType.DMA,
        pltpu.SemaphoreType.DMA,
        pltpu.SemaphoreType.DMA,
        pltpu.SemaphoreType.DMA,
    ],
    compiler_params=pltpu.CompilerParams(use_tc_tiling_on_sc=False,
                                         needs_layout_passes=False),
)(_sc_body)


def _mm_body(w_ref, x_ref, o_ref):
    zt = lax.dot_general(w_ref[...], x_ref[...], (((0,), (1,)), ((), ())),
                         preferred_element_type=jnp.float32)
    o_ref[...] = jnp.concatenate(
        [zt[0:32, :], zt[32:64, :], zt[64:96, :], zt[96:128, :]], axis=1)


def _mm_body_alias(w_ref, x_ref, prev_ref, o_ref):
    del prev_ref
    _mm_body(w_ref, x_ref, o_ref)


def _matmul_first(w_big, feats_packed):
    return pl.pallas_call(
        _mm_body,
        grid=(_NSBH,),
        in_specs=[pl.BlockSpec((128, 128), lambda i: (0, 0)),
                  pl.BlockSpec((_TBS, 128), lambda i: (i, 0))],
        out_specs=pl.BlockSpec((_OUT_DIM, _SBE), lambda i: (0, i)),
        out_shape=jax.ShapeDtypeStruct((_OUT_DIM, _N_EDGES), jnp.float32),
    )(w_big, feats_packed)


def _matmul_second(w_big, feats_packed, prev):
    return pl.pallas_call(
        _mm_body_alias,
        grid=(_NSBH,),
        in_specs=[pl.BlockSpec((128, 128), lambda i: (0, 0)),
                  pl.BlockSpec((_TBS, 128), lambda i: (i, 0)),
                  pl.BlockSpec(memory_space=pltpu.HBM)],
        out_specs=pl.BlockSpec((_OUT_DIM, _SBE), lambda i: (0, i + _NSBH)),
        out_shape=jax.ShapeDtypeStruct((_OUT_DIM, _N_EDGES), jnp.float32),
        input_output_aliases={2: 0},
    )(w_big, feats_packed, prev)


def kernel(frame_t, frame_R, edge_src, edge_dst, W):
    table = jnp.concatenate(
        [frame_t, frame_R.reshape(_N_NODES, 9),
         jnp.zeros((_N_NODES, 4), jnp.float32)], axis=1)
    w_big = jnp.zeros((128, 128), jnp.float32)
    for j in range(4):
        w_big = w_big.at[j * 32:j * 32 + 31, j * 32:(j + 1) * 32].set(W)
    esrc = edge_src.astype(jnp.int32)
    edst = edge_dst.astype(jnp.int32)
    f1 = _sc_feats(table, esrc[:_EH], edst[:_EH])
    f2 = _sc_feats(table, esrc[_EH:], edst[_EH:])
    o1 = _matmul_first(w_big, f1)
    out_t = _matmul_second(w_big, f2, o1)
    return out_t.T


# speedup vs baseline: 44.5561x; 1.0004x over previous
import functools
import math

import jax
import jax.numpy as jnp
from jax import lax
from jax.experimental import pallas as pl
from jax.experimental.pallas import tpu as pltpu
from jax.experimental.pallas import tpu_sc as plsc

_N_NODES = 100000
_N_EDGES = 1600000
_N_BASIS = 16
_OUT_DIM = 32
_D_MAX = 20.0

_NC = 2
_NS = 16
_NW = _NC * _NS
_L = 16

_BLK = 400
_SUB = 80
_NSUB = _BLK // _SUB
_GRP = _BLK // _L
_TW = 16

_SBE = 6400
_NSB = _N_EDGES // _SBE
_TBS = _SBE // 4
_EH = _N_EDGES // 2
_NSBH = _EH // _SBE


def _rsqrt(a):
    i = plsc.bitcast(a, jnp.int32)
    y = plsc.bitcast(jnp.int32(0x5F3759DF) - (i >> 1), jnp.float32)
    h = 0.5 * a
    for _ in range(2):
        y = y * (1.5 - (h * y) * y)
    return y


def _sc_body(table, esrc, edst, feats,
             isv0, idv0, isv1, idv1,
             rs0, rd0, rs1, rd1,
             fb0, fb1,
             s_idx0, s_idx1, sgs0, sgd0, sgs1, sgd1, so0, so1):
    n_sb = _NSBH
    q, rm = divmod(n_sb, _NW)
    cid = lax.axis_index("c")
    sid = lax.axis_index("s")
    wid = sid * _NC + cid
    sb_start = jnp.where(wid < rm, (q + 1) * wid, q * wid + rm)
    nblk = jnp.where(wid < rm, (q + 1) * _SBE // _BLK, q * _SBE // _BLK)
    npair = nblk // 2
    wbase = sb_start * _SBE
    ii = lax.iota(jnp.int32, _L)
    zero = jnp.zeros((_L,), jnp.float32)
    cols = [jnp.full((_L,), f, jnp.int32) for f in range(_OUT_DIM)]
    beta = _D_MAX / (_N_BASIS - 1)
    kconst = [jnp.float32(math.exp(-beta * beta * (2 * b + 1)))
              for b in range(_N_BASIS - 1)]

    def fire_idx(iref, dref, sem, blk):
        ebase = wbase + blk * _BLK
        pltpu.async_copy(esrc.at[pl.ds(ebase, _BLK)], iref, sem)
        pltpu.async_copy(edst.at[pl.ds(ebase, _BLK)], dref, sem)

    def wait_idx(iref, dref, sem):
        pltpu.make_async_copy(esrc.at[pl.ds(0, _BLK)], iref, sem).wait()
        pltpu.make_async_copy(esrc.at[pl.ds(0, _BLK)], dref, sem).wait()

    def fire_gathers(iref, dref, rs, rd, sgs, sgd):
        for k in range(_NSUB):
            sl = pl.ds(k * _SUB, _SUB)
            pltpu.async_copy(table.at[iref.at[sl]], rs.at[sl], sgs)
            pltpu.async_copy(table.at[dref.at[sl]], rd.at[sl], sgd)

    def wait_gathers(rs, rd, sgs, sgd):
        pltpu.make_async_copy(table.at[pl.ds(0, _BLK)], rs, sgs).wait()
        pltpu.make_async_copy(table.at[pl.ds(0, _BLK)], rd, sgd).wait()

    def fire_out(fb, sem, blk):
        eb = wbase + blk * _BLK
        sb = eb // _SBE
        off = eb % _SBE
        jb = off // _TBS
        r0 = pl.multiple_of(sb * _TBS + off % _TBS, _BLK)
        pltpu.async_copy(
            fb.at[pl.ds(0, _BLK), pl.ds(0, 32)],
            feats.at[pl.ds(r0, _BLK),
                     pl.ds(pl.multiple_of(jb * 32, 32), 32)], sem)

    def wait_out(fb, sem):
        pltpu.make_async_copy(
            fb.at[pl.ds(0, _BLK), pl.ds(0, 32)],
            feats.at[pl.ds(0, _BLK), pl.ds(0, 32)], sem).wait()

    def compute_block(rs, rd, fb):
        def grp(g, carry):
            rows = g * _L + ii
            s = [plsc.load_gather(rs, [rows, cols[f]]) for f in range(12)]
            d = [plsc.load_gather(rd, [rows, cols[f]]) for f in range(12)]
            dx = d[0] - s[0]
            dy = d[1] - s[1]
            dz = d[2] - s[2]
            d2 = dx * dx + dy * dy + dz * dz
            dist = d2 * _rsqrt(d2)
            inv_den = _rsqrt(d2 + 1.0)
            Rs = s[3:]
            Rd = d[3:]
            fv = []
            r = jnp.exp(-d2)
            u = jnp.exp((2.0 * beta) * dist)
            fv.append(r)
            for kc in kconst:
                r = r * (u * kc)
                fv.append(r)
            for k in range(3):
                fv.append((dx * Rs[k] + dy * Rs[3 + k] + dz * Rs[6 + k]) * inv_den)
            ninv = -inv_den
            for k in range(3):
                fv.append((dx * Rd[k] + dy * Rd[3 + k] + dz * Rd[6 + k]) * ninv)
            for a in range(3):
                for k in range(3):
                    fv.append(Rs[a] * Rd[k] + Rs[3 + a] * Rd[3 + k]
                              + Rs[6 + a] * Rd[6 + k])
            fv.append(zero)
            for f, v in enumerate(fv):
                plsc.store_scatter(fb, [rows, cols[f]], v)
            return carry

        lax.fori_loop(0, _GRP, grp, 0)

    fire_idx(isv0, idv0, s_idx0, 0)
    fire_idx(isv1, idv1, s_idx1, 1)
    wait_idx(isv0, idv0, s_idx0)
    fire_gathers(isv0, idv0, rs0, rd0, sgs0, sgd0)

    def pair(k, carry):
        a = 2 * k
        b = a + 1
        wait_idx(isv1, idv1, s_idx1)
        fire_gathers(isv1, idv1, rs1, rd1, sgs1, sgd1)
        wait_gathers(rs0, rd0, sgs0, sgd0)

        @pl.when(k < npair - 1)
        def _():
            fire_idx(isv0, idv0, s_idx0, a + 2)

        @pl.when(k >= 1)
        def _():
            wait_out(fb0, so0)

        compute_block(rs0, rd0, fb0)
        fire_out(fb0, so0, a)

        @pl.when(k < npair - 1)
        def _():
            wait_idx(isv0, idv0, s_idx0)
            fire_gathers(isv0, idv0, rs0, rd0, sgs0, sgd0)

        wait_gathers(rs1, rd1, sgs1, sgd1)

        @pl.when(k < npair - 1)
        def _():
            fire_idx(isv1, idv1, s_idx1, b + 2)

        @pl.when(k >= 1)
        def _():
            wait_out(fb1, so1)

        compute_block(rs1, rd1, fb1)
        fire_out(fb1, so1, b)
        return carry

    lax.fori_loop(0, npair, pair, 0)

    wait_out(fb0, so0)
    wait_out(fb1, so1)


_sc_feats = functools.partial(
    pl.kernel,
    out_type=jax.ShapeDtypeStruct((_EH // 4, 128), jnp.float32),
    mesh=plsc.VectorSubcoreMesh(core_axis_name="c", subcore_axis_name="s",
                                num_cores=_NC, num_subcores=_NS),
    scratch_types=[
        pltpu.VMEM((_BLK,), jnp.int32),
        pltpu.VMEM((_BLK,), jnp.int32),
        pltpu.VMEM((_BLK,), jnp.int32),
        pltpu.VMEM((_BLK,), jnp.int32),
        pltpu.VMEM((_BLK, _TW), jnp.float32),
        pltpu.VMEM((_BLK, _TW), jnp.float32),
        pltpu.VMEM((_BLK, _TW), jnp.float32),
        pltpu.VMEM((_BLK, _TW), jnp.float32),
        pltpu.VMEM((_BLK, 33), jnp.float32),
        pltpu.VMEM((_BLK, 33), jnp.float32),
        pltpu.SemaphoreType.DMA,
        pltpu.SemaphoreType.DMA,
        pltpu.SemaphoreType.DMA,
        pltpu.SemaphoreType.DMA,
        pltpu.SemaphoreType.DMA,
        pltpu.SemaphoreType.DMA,
        pltpu.SemaphoreType.DMA,
        pltpu.SemaphoreType.DMA,
    ],
    compiler_params=pltpu.CompilerParams(use_tc_tiling_on_sc=False,
                                         needs_layout_passes=False),
)(_sc_body)


def _mm_body(w_ref, x_ref, o_ref):
    zt = lax.dot_general(w_ref[...], x_ref[...], (((0,), (1,)), ((), ())),
                         preferred_element_type=jnp.float32)
    o_ref[...] = jnp.concatenate(
        [zt[0:32, :], zt[32:64, :], zt[64:96, :], zt[96:128, :]], axis=1)


def _mm_body_alias(w_ref, x_ref, prev_ref, o_ref):
    del prev_ref
    _mm_body(w_ref, x_ref, o_ref)


def _matmul_first(w_big, feats_packed):
    return pl.pallas_call(
        _mm_body,
        grid=(_NSBH,),
        in_specs=[pl.BlockSpec((128, 128), lambda i: (0, 0)),
                  pl.BlockSpec((_TBS, 128), lambda i: (i, 0))],
        out_specs=pl.BlockSpec((_OUT_DIM, _SBE), lambda i: (0, i)),
        out_shape=jax.ShapeDtypeStruct((_OUT_DIM, _N_EDGES), jnp.float32),
    )(w_big, feats_packed)


def _matmul_second(w_big, feats_packed, prev):
    return pl.pallas_call(
        _mm_body_alias,
        grid=(_NSBH,),
        in_specs=[pl.BlockSpec((128, 128), lambda i: (0, 0)),
                  pl.BlockSpec((_TBS, 128), lambda i: (i, 0)),
                  pl.BlockSpec(memory_space=pltpu.HBM)],
        out_specs=pl.BlockSpec((_OUT_DIM, _SBE), lambda i: (0, i + _NSBH)),
        out_shape=jax.ShapeDtypeStruct((_OUT_DIM, _N_EDGES), jnp.float32),
        input_output_aliases={2: 0},
    )(w_big, feats_packed, prev)


def kernel(frame_t, frame_R, edge_src, edge_dst, W):
    table = jnp.concatenate(
        [frame_t, frame_R.reshape(_N_NODES, 9),
         jnp.zeros((_N_NODES, _TW - 12), jnp.float32)], axis=1)
    w_big = jnp.zeros((128, 128), jnp.float32)
    for j in range(4):
        w_big = w_big.at[j * 32:j * 32 + 31, j * 32:(j + 1) * 32].set(W)
    esrc = edge_src.astype(jnp.int32)
    edst = edge_dst.astype(jnp.int32)
    f1 = _sc_feats(table, esrc[:_EH], edst[:_EH])
    f2 = _sc_feats(table, esrc[_EH:], edst[_EH:])
    o1 = _matmul_first(w_big, f1)
    out_t = _matmul_second(w_big, f2, o1)
    return out_t.T
